# baseline (speedup 1.0000x reference)
import functools

import jax
import jax.numpy as jnp
from jax import lax
from jax.experimental import pallas as pl
from jax.experimental.pallas import tpu as pltpu
from jax.experimental.pallas import tpu_sc as plsc

NUM_EXPERTS = 3
IN_FEATURES = 1024
OUT_FEATURES = 1024
N_TOKENS = 8192

BLK = 512
NP = N_TOKENS + NUM_EXPERTS * BLK
NB = NP // BLK

_INFO = plsc.get_sparse_core_info()
NC, NS = _INFO.num_cores, _INFO.num_subcores
NW = NC * NS
PER_W = N_TOKENS // NW
CH = 32
NCH = PER_W // CH

_mesh = plsc.VectorSubcoreMesh(core_axis_name="c", subcore_axis_name="s")


def _routing_body(ids_ref, dest_ref, be_ref):
    ids = ids_ref[...].astype(jnp.float32)
    rows, cols = ids.shape
    ui = lax.broadcasted_iota(jnp.int32, (cols, cols), 0)
    uj = lax.broadcasted_iota(jnp.int32, (cols, cols), 1)
    upper = (ui <= uj).astype(jnp.float32)
    si = lax.broadcasted_iota(jnp.int32, (rows, rows), 0)
    sj = lax.broadcasted_iota(jnp.int32, (rows, rows), 1)
    strict = (sj < si).astype(jnp.float32)

    counts = []
    ranks = []
    for e in range(NUM_EXPERTS):
        m = (ids == float(e)).astype(jnp.float32)
        rowcum = jax.lax.dot_general(
            m, upper, dimension_numbers=(((1,), (0,)), ((), ())),
            preferred_element_type=jnp.float32,
            precision=jax.lax.Precision.HIGHEST,
        )
        rowsum = rowcum[:, cols - 1:cols]
        rowpref = jax.lax.dot_general(
            strict, rowsum, dimension_numbers=(((1,), (0,)), ((), ())),
            preferred_element_type=jnp.float32,
            precision=jax.lax.Precision.HIGHEST,
        )
        ranks.append(rowcum - m + rowpref)
        counts.append(jnp.sum(m))

    seg = [jnp.float32(0.0)] * (NUM_EXPERTS + 1)
    for e in range(NUM_EXPERTS):
        padded = jnp.ceil(counts[e] / BLK) * BLK
        seg[e + 1] = seg[e] + padded

    dest = jnp.zeros_like(ids)
    for e in range(NUM_EXPERTS):
        m = (ids == float(e)).astype(jnp.float32)
        dest = dest + m * (seg[e] + ranks[e])
    dest_ref[...] = dest.astype(jnp.int32)

    bstart = lax.broadcasted_iota(jnp.int32, (1, 128), 1).astype(jnp.float32) * float(BLK)
    be = jnp.zeros((1, 128), jnp.float32)
    for e in range(1, NUM_EXPERTS):
        be = be + (bstart >= seg[e]).astype(jnp.float32)
    be_ref[...] = be.astype(jnp.int32)


def _routing(ids2d):
    return pl.pallas_call(
        _routing_body,
        in_specs=[pl.BlockSpec(ids2d.shape, lambda: (0, 0))],
        out_specs=[
            pl.BlockSpec(ids2d.shape, lambda: (0, 0)),
            pl.BlockSpec((1, 128), lambda: (0, 0)),
        ],
        out_shape=[
            jax.ShapeDtypeStruct(ids2d.shape, jnp.int32),
            jax.ShapeDtypeStruct((1, 128), jnp.int32),
        ],
    )(ids2d)


@functools.partial(
    pl.kernel,
    mesh=_mesh,
    out_type=jax.ShapeDtypeStruct((NP, IN_FEATURES), jnp.float32),
    scratch_types=[
        pltpu.VMEM((CH,), jnp.int32),
        pltpu.VMEM((CH,), jnp.int32),
        pltpu.VMEM((CH, IN_FEATURES), jnp.float32),
        pltpu.VMEM((CH, IN_FEATURES), jnp.float32),
        pltpu.SemaphoreType.DMA,
    ],
)
def _scatter_x(x_hbm, dest_hbm, xs_hbm, idx0, idx1, rows0, rows1, sem):
    wid = lax.axis_index("s") * NC + lax.axis_index("c")
    base = wid * PER_W
    idxs = [idx0, idx1]
    rows = [rows0, rows1]
    pltpu.sync_copy(dest_hbm.at[pl.ds(base, CH)], idx0)
    pltpu.sync_copy(x_hbm.at[pl.ds(base, CH)], rows0)
    h = pltpu.async_copy(rows0, xs_hbm.at[idx0], sem)
    for c in range(NCH):
        b = c & 1
        nxt = (c + 1) & 1
        if c + 1 < NCH:
            o = base + (c + 1) * CH
            pltpu.sync_copy(dest_hbm.at[pl.ds(o, CH)], idxs[nxt])
            pltpu.sync_copy(x_hbm.at[pl.ds(o, CH)], rows[nxt])
            hn = pltpu.async_copy(rows[nxt], xs_hbm.at[idxs[nxt]], sem)
        h.wait()
        if c + 1 < NCH:
            h = hn


def _mm_body(be_ref, xs_ref, w_ref, out_ref):
    del be_ref
    out_ref[...] = jax.lax.dot_general(
        xs_ref[...], w_ref[0],
        dimension_numbers=(((1,), (1,)), ((), ())),
        preferred_element_type=jnp.float32,
    )


def _grouped_mm(be, xs, w):
    grid_spec = pltpu.PrefetchScalarGridSpec(
        num_scalar_prefetch=1,
        grid=(NB,),
        in_specs=[
            pl.BlockSpec((BLK, IN_FEATURES), lambda i, be: (i, 0)),
            pl.BlockSpec(
                (1, OUT_FEATURES, IN_FEATURES), lambda i, be: (be[i], 0, 0)
            ),
        ],
        out_specs=pl.BlockSpec((BLK, OUT_FEATURES), lambda i, be: (i, 0)),
    )
    return pl.pallas_call(
        _mm_body,
        grid_spec=grid_spec,
        out_shape=jax.ShapeDtypeStruct((NP, OUT_FEATURES), jnp.float32),
    )(be, xs, w)


@functools.partial(
    pl.kernel,
    mesh=_mesh,
    out_type=jax.ShapeDtypeStruct((N_TOKENS, OUT_FEATURES), jnp.float32),
    scratch_types=[
        pltpu.VMEM((CH,), jnp.int32),
        pltpu.VMEM((CH,), jnp.int32),
        pltpu.VMEM((CH, OUT_FEATURES), jnp.float32),
        pltpu.VMEM((CH, OUT_FEATURES), jnp.float32),
        pltpu.SemaphoreType.DMA,
    ],
)
def _gather_out(os_hbm, dest_hbm, out_hbm, idx0, idx1, rows0, rows1, sem):
    wid = lax.axis_index("s") * NC + lax.axis_index("c")
    base = wid * PER_W
    idxs = [idx0, idx1]
    rows = [rows0, rows1]
    pltpu.sync_copy(dest_hbm.at[pl.ds(base, CH)], idx0)
    h = pltpu.async_copy(os_hbm.at[idx0], rows0, sem)
    for c in range(NCH):
        b = c & 1
        nxt = (c + 1) & 1
        if c + 1 < NCH:
            o = base + (c + 1) * CH
            pltpu.sync_copy(dest_hbm.at[pl.ds(o, CH)], idxs[nxt])
            hn = pltpu.async_copy(os_hbm.at[idxs[nxt]], rows[nxt], sem)
        h.wait()
        pltpu.sync_copy(rows[b], out_hbm.at[pl.ds(base + c * CH, CH)])
        if c + 1 < NCH:
            h = hn


def kernel(x, modality_ids, weight):
    w = weight.reshape(NUM_EXPERTS, OUT_FEATURES, IN_FEATURES)
    ids2d = modality_ids.astype(jnp.int32).reshape(64, 128)
    dest2d, be2d = _routing(ids2d)
    dest = dest2d.reshape(N_TOKENS)
    be = be2d.reshape(128)[:NB]
    xs = _scatter_x(x, dest)
    os = _grouped_mm(be, xs, w)
    return _gather_out(os, dest)


# speedup vs baseline: 1.7363x; 1.7363x over previous
import jax
import jax.numpy as jnp
from jax.experimental import pallas as pl

NUM_EXPERTS = 3
IN_FEATURES = 1024
OUT_FEATURES = 1024
N_TOKENS = 8192
TOKEN_BLOCK = 1024


def _body(x_ref, ids_ref, w_ref, out_ref):
    x = x_ref[...]
    ids = ids_ref[...]
    acc = jnp.zeros((x.shape[0], OUT_FEATURES), jnp.float32)
    for e in range(NUM_EXPERTS):
        y = jax.lax.dot_general(
            x, w_ref[e],
            dimension_numbers=(((1,), (1,)), ((), ())),
            preferred_element_type=jnp.float32,
        )
        acc = jnp.where(ids == float(e), y, acc)
    out_ref[...] = acc


def kernel(x, modality_ids, weight):
    w = weight.reshape(NUM_EXPERTS, OUT_FEATURES, IN_FEATURES)
    ids_f = modality_ids.astype(jnp.float32).reshape(N_TOKENS, 1)
    nb = N_TOKENS // TOKEN_BLOCK
    return pl.pallas_call(
        _body,
        grid=(nb,),
        in_specs=[
            pl.BlockSpec((TOKEN_BLOCK, IN_FEATURES), lambda i: (i, 0)),
            pl.BlockSpec((TOKEN_BLOCK, 1), lambda i: (i, 0)),
            pl.BlockSpec(
                (NUM_EXPERTS, OUT_FEATURES, IN_FEATURES), lambda i: (0, 0, 0)
            ),
        ],
        out_specs=pl.BlockSpec((TOKEN_BLOCK, OUT_FEATURES), lambda i: (i, 0)),
        out_shape=jax.ShapeDtypeStruct((N_TOKENS, OUT_FEATURES), jnp.float32),
    )(x, ids_f, w)
